# baseline (speedup 1.0000x reference)
import functools
import jax
import jax.numpy as jnp
from jax import lax
from jax.experimental import pallas as pl
from jax.experimental.pallas import tpu as pltpu
from jax.experimental.pallas import tpu_sc as plsc

N = 10000
NP = 10240
E = 320000
HC = 128
NH = 8
HD = 16
NC = 2
NS = 16
NW = NC * NS
NBUF = 5
K = 80
BN = 400


def _dense_pre_body(p_ref, w_ref, as_ref, ad_ref, h_ref, oas_ref, oad_ref):
    h = jnp.dot(p_ref[...], w_ref[...], preferred_element_type=jnp.float32)
    h_ref[...] = h
    a_s = (h * as_ref[...]).reshape(BN, NH, HD).sum(-1)
    a_d = (h * ad_ref[...]).reshape(BN, NH, HD).sum(-1)
    oas_ref[...] = jnp.concatenate([a_s, a_s], axis=1)
    oad_ref[...] = jnp.concatenate([a_d, a_d], axis=1)


def _dense_pre(p, w, as2, ad2):
    grid = (N // BN,)
    return pl.pallas_call(
        _dense_pre_body,
        grid=grid,
        in_specs=[
            pl.BlockSpec((BN, HC), lambda i: (i, 0)),
            pl.BlockSpec((HC, HC), lambda i: (0, 0)),
            pl.BlockSpec((1, HC), lambda i: (0, 0)),
            pl.BlockSpec((1, HC), lambda i: (0, 0)),
        ],
        out_specs=[
            pl.BlockSpec((BN, HC), lambda i: (i, 0)),
            pl.BlockSpec((BN, 16), lambda i: (i, 0)),
            pl.BlockSpec((BN, 16), lambda i: (i, 0)),
        ],
        out_shape=[
            jax.ShapeDtypeStruct((NP, HC), jnp.float32),
            jax.ShapeDtypeStruct((NP, 16), jnp.float32),
            jax.ShapeDtypeStruct((NP, 16), jnp.float32),
        ],
    )(p, w, as2, ad2)


def _expand_heads(v8, rows):
    return jnp.broadcast_to(v8[:, :, None], (rows, NH, HD)).reshape(rows, HC)


def _finalize_body(self_loops, acc_ref, den_ref, h_ref, as_ref, ad_ref,
                   loop_ref, c_ref, b_ref, out_ref):
    acc = acc_ref[0] + acc_ref[1]
    den16 = den_ref[0] + den_ref[1]
    if self_loops:
        ls = loop_ref[0] + loop_ref[1]
        ea_mean = ls[:, 8:9] / jnp.maximum(ls[:, 9:10], 1.0)
        al = as_ref[...] + ad_ref[...] + ea_mean * c_ref[...]
        al = jnp.maximum(al, 0.2 * al)
        ws = jnp.exp(al)
        den16 = den16 + ws
        acc = acc + h_ref[...] * _expand_heads(ws[:, :NH], BN)
    den = _expand_heads(den16[:, :NH], BN)
    out_ref[...] = jnp.maximum(acc / (den + 1e-16) + b_ref[...], 0.0)


def _finalize(acc2, den2, h, as2, ad2, loop2, c16, bias, self_loops):
    grid = (N // BN,)
    body = functools.partial(_finalize_body, self_loops)
    return pl.pallas_call(
        body,
        grid=grid,
        in_specs=[
            pl.BlockSpec((2, BN, HC), lambda i: (0, i, 0)),
            pl.BlockSpec((2, BN, 16), lambda i: (0, i, 0)),
            pl.BlockSpec((BN, HC), lambda i: (i, 0)),
            pl.BlockSpec((BN, 16), lambda i: (i, 0)),
            pl.BlockSpec((BN, 16), lambda i: (i, 0)),
            pl.BlockSpec((2, BN, 16), lambda i: (0, i, 0)),
            pl.BlockSpec((1, 16), lambda i: (0, 0)),
            pl.BlockSpec((1, HC), lambda i: (0, 0)),
        ],
        out_specs=pl.BlockSpec((BN, HC), lambda i: (i, 0)),
        out_shape=jax.ShapeDtypeStruct((N, HC), jnp.float32),
    )(acc2, den2, h, as2, ad2, loop2, c16, bias)


def _edge_body(src_hbm, dst_hbm, ea_hbm, c_hbm, as_hbm, ad_hbm,
               h_hbm, z128_hbm, z16_hbm,
               acc_out, den_out,
               srcb, dstb, eab, c_v, as_r, ad_r, h_r, w_den,
               acc_sh, den_sh, sem_i, sem_g, sem_s):
    s = lax.axis_index("s")
    c = lax.axis_index("c")
    wid = s * NC + c
    rows_per = NP // NS
    rbase = s * rows_per
    pltpu.sync_copy(z128_hbm.at[pl.ds(rbase, rows_per)],
                    acc_sh.at[pl.ds(rbase, rows_per)])
    pltpu.sync_copy(z16_hbm.at[pl.ds(rbase, rows_per)],
                    den_sh.at[pl.ds(rbase, rows_per)])
    plsc.subcore_barrier()
    pltpu.sync_copy(c_hbm, c_v)
    cvec = c_v[...]
    ew = E // NW
    ebase = wid * ew
    niter = ew // K
    lanes = lax.iota(jnp.int32, 16)

    def issue_idx(it, b):
        off = ebase + it * K
        pltpu.async_copy(src_hbm.at[pl.ds(off, K)], srcb.at[b], sem_i.at[b])
        pltpu.async_copy(dst_hbm.at[pl.ds(off, K)], dstb.at[b], sem_i.at[b])
        pltpu.async_copy(ea_hbm.at[pl.ds(off, K)], eab.at[b], sem_i.at[b])

    def wait_idx(b):
        pltpu.make_async_copy(src_hbm.at[pl.ds(0, K)], srcb.at[b],
                              sem_i.at[b]).wait()
        pltpu.make_async_copy(dst_hbm.at[pl.ds(0, K)], dstb.at[b],
                              sem_i.at[b]).wait()
        pltpu.make_async_copy(ea_hbm.at[pl.ds(0, K)], eab.at[b],
                              sem_i.at[b]).wait()

    def issue_gather_ad(b4, b2):
        pltpu.async_copy(as_hbm.at[srcb.at[b4]], as_r.at[b2], sem_g.at[b2])
        pltpu.async_copy(ad_hbm.at[dstb.at[b4]], ad_r.at[b2], sem_g.at[b2])

    def issue_gather_h(b4, b2):
        pltpu.async_copy(h_hbm.at[srcb.at[b4]], h_r.at[b2], sem_g.at[b2])

    def wait_gather(b4, b2):
        pltpu.make_async_copy(as_hbm.at[srcb.at[b4]], as_r.at[b2],
                              sem_g.at[b2]).wait()
        pltpu.make_async_copy(ad_hbm.at[dstb.at[b4]], ad_r.at[b2],
                              sem_g.at[b2]).wait()
        pltpu.make_async_copy(h_hbm.at[srcb.at[b4]], h_r.at[b2],
                              sem_g.at[b2]).wait()

    def issue_scatter(b4, b2):
        pltpu.async_copy(w_den.at[b4], den_sh.at[dstb.at[b4]], sem_s.at[b4],
                         add=True)
        pltpu.async_copy(h_r.at[b2], acc_sh.at[dstb.at[b4]], sem_s.at[b4],
                         add=True)

    def wait_scatter(b4, b2):
        pltpu.make_async_copy(w_den.at[b4], den_sh.at[dstb.at[b4]],
                              sem_s.at[b4]).wait()
        pltpu.make_async_copy(h_r.at[b2], acc_sh.at[dstb.at[b4]],
                              sem_s.at[b4]).wait()

    dnums = lax.GatherDimensionNumbers(offset_dims=(),
                                        collapsed_slice_dims=(0,),
                                        start_index_map=(0,))

    def bcast(v, lane):
        idx = (lanes * 0 + lane).reshape(16, 1)
        return lax.gather(v, idx, dnums, (1,),
                          mode=lax.GatherScatterMode.PROMISE_IN_BOUNDS)

    def compute(b4, b2):
        def blk16(jb, carry2):
            eb = eab[b4, pl.ds(jb * 16, 16)]
            for i in range(16):
                j = jb * 16 + i
                eas = eb[i]
                eav = bcast(eb, i)
                al = as_r[b2, j] + ad_r[b2, j] + eav * cvec
                al = jnp.maximum(al, 0.2 * al)
                w = jnp.exp(al)
                w_den[b4, j] = jnp.where(
                    lanes < 8, w,
                    jnp.where(lanes == 8, eas,
                              jnp.where(lanes == 9, 1.0, 0.0)))
                for hh in range(NH):
                    sl = pl.ds(hh * HD, HD)
                    h_r[b2, j, sl] = h_r[b2, j, sl] * bcast(w, hh)
            return carry2

        lax.fori_loop(0, K // 16, blk16, 0)

    def body(it, b0):
        b2 = b0 % 2
        bp1_4 = (b0 + 1) % 4
        bp1_2 = (b0 + 1) % 2
        bp2_4 = (b0 + 2) % 4

        @pl.when(it + 2 < niter)
        def _():
            issue_idx(it + 2, bp2_4)

        @pl.when(it + 1 < niter)
        def _():
            wait_idx(bp1_4)
            issue_gather_ad(bp1_4, bp1_2)

        wait_gather(b0 % 4, b2)
        compute(b0 % 4, b2)

        @pl.when(it >= 1)
        def _():
            wait_scatter((b0 + 3) % 4, bp1_2)

        @pl.when(it + 1 < niter)
        def _():
            issue_gather_h(bp1_4, bp1_2)

        issue_scatter(b0 % 4, b2)

    issue_idx(0, 0)
    issue_idx(1, 1)
    wait_idx(0)
    issue_gather_ad(0, 0)
    issue_gather_h(0, 0)

    def outer(g, carry):
        for b0 in range(4):
            body(g * 4 + b0, b0)
        return carry

    lax.fori_loop(0, niter // 4, outer, 0)
    body(niter - 1, 0)
    wait_scatter(0, 0)
    plsc.subcore_barrier()
    obase = c * NP + rbase
    pltpu.sync_copy(acc_sh.at[pl.ds(rbase, rows_per)],
                    acc_out.at[pl.ds(obase, rows_per)])
    pltpu.sync_copy(den_sh.at[pl.ds(rbase, rows_per)],
                    den_out.at[pl.ds(obase, rows_per)])


def _edge_pass(src, dst, ea, c16, as2, ad2, h, z128, z16):
    mesh = plsc.VectorSubcoreMesh(core_axis_name="c", subcore_axis_name="s",
                                  num_cores=NC, num_subcores=NS)
    out_type = [
        jax.ShapeDtypeStruct((NC * NP, HC), jnp.float32),
        jax.ShapeDtypeStruct((NC * NP, 16), jnp.float32),
    ]
    scratch = [
        pltpu.VMEM((4, K), jnp.int32),
        pltpu.VMEM((4, K), jnp.int32),
        pltpu.VMEM((4, K), jnp.float32),
        pltpu.VMEM((16,), jnp.float32),
        pltpu.VMEM((2, K, 16), jnp.float32),
        pltpu.VMEM((2, K, 16), jnp.float32),
        pltpu.VMEM((2, K, HC), jnp.float32),
        pltpu.VMEM((4, K, 16), jnp.float32),
        pltpu.VMEM_SHARED((NP, HC), jnp.float32),
        pltpu.VMEM_SHARED((NP, 16), jnp.float32),
        pltpu.SemaphoreType.DMA((4,)),
        pltpu.SemaphoreType.DMA((2,)),
        pltpu.SemaphoreType.DMA((4,)),
    ]
    fn = pl.kernel(_edge_body, out_type=out_type, mesh=mesh,
                   scratch_types=scratch,
                   compiler_params=pltpu.CompilerParams(
                       use_tc_tiling_on_sc=False))
    acc2, den2 = fn(src, dst, ea, c16, as2, ad2, h, z128, z16)
    return acc2.reshape(NC, NP, HC), den2.reshape(NC, NP, 16)


def _sigmoid(x):
    return 1.0 / (1.0 + jnp.exp(-x))


def _head_body(na, g, hs, ad, h_ref, b_ref, hid_ref, wih_ref, whh_ref,
               bih_ref, bhh_ref, f1w_ref, f1b_ref, f2w_ref, f2b_ref,
               mean_ref, hl_ref, agent_ref):
    gvec = lax.broadcasted_iota(jnp.int32, (1, g), 1).astype(jnp.float32)
    cb = 200

    def blk(i, carry):
        sums, cnt, ltc = carry
        bblk = b_ref[pl.ds(i * cb, cb), :]
        hblk = h_ref[pl.ds(i * cb, cb), :]
        oh = jnp.where(bblk == gvec, 1.0, 0.0)
        lt = jnp.where(bblk < gvec, 1.0, 0.0)
        sums = sums + lax.dot_general(oh, hblk, (((0,), (0,)), ((), ())),
                                      preferred_element_type=jnp.float32)
        cnt = cnt + jnp.sum(oh, axis=0)
        ltc = ltc + jnp.sum(lt, axis=0)
        return sums, cnt, ltc

    sums, cnt, ltc = lax.fori_loop(
        0, N // cb, blk,
        (jnp.zeros((g, HC), jnp.float32), jnp.zeros((g,), jnp.float32),
         jnp.zeros((g,), jnp.float32)))
    ge = sums / jnp.maximum(cnt, 1.0)[:, None]
    starts = ltc.astype(jnp.int32)
    for gg in range(g):
        sg = starts[gg]
        for a in range(na):
            idx = jnp.minimum(sg + a, N - 1)
            agent_ref[pl.ds(a * g + gg, 1), :] = h_ref[pl.ds(idx, 1), :]
    hcur = hid_ref[...]
    ys = []
    for t in range(na):
        xa = agent_ref[pl.ds(t * g, g), :]
        xt = jnp.concatenate([xa, ge], axis=1)
        gi = lax.dot_general(xt, wih_ref[...], (((1,), (1,)), ((), ())),
                             preferred_element_type=jnp.float32) + bih_ref[...]
        gh = lax.dot_general(hcur, whh_ref[...], (((1,), (1,)), ((), ())),
                             preferred_element_type=jnp.float32) + bhh_ref[...]
        r = _sigmoid(gi[:, :hs] + gh[:, :hs])
        z = _sigmoid(gi[:, hs:2 * hs] + gh[:, hs:2 * hs])
        nt = jnp.tanh(gi[:, 2 * hs:] + r * gh[:, 2 * hs:])
        hcur = (1.0 - z) * nt + z * hcur
        ys.append(hcur)
    gru = jnp.stack(ys, axis=1).reshape(g * na, hs)
    f1 = jnp.maximum(
        lax.dot_general(gru, f1w_ref[...], (((1,), (1,)), ((), ())),
                        preferred_element_type=jnp.float32) + f1b_ref[...],
        0.0)
    mo = lax.dot_general(f1, f2w_ref[...], (((1,), (1,)), ((), ())),
                         preferred_element_type=jnp.float32) + f2b_ref[...]
    mean_ref[...] = mo
    hl_ref[...] = hcur


def _head(h, batch_f, hidden, p, g, na, hs, ad):
    body = functools.partial(_head_body, na, g, hs, ad)
    return pl.pallas_call(
        body,
        out_shape=[
            jax.ShapeDtypeStruct((g * na, ad), jnp.float32),
            jax.ShapeDtypeStruct((g, hs), jnp.float32),
        ],
        scratch_shapes=[pltpu.VMEM((g * na, HC), jnp.float32)],
    )(h, batch_f, hidden, p["w_ih"], p["w_hh"], p["b_ih"].reshape(1, -1),
      p["b_hh"].reshape(1, -1), p["fc1_w"], p["fc1_b"].reshape(1, -1),
      p["fc2_w"], p["fc2_b"].reshape(1, -1))


def _layer_c16(le, ae):
    c = (le.reshape(NH, HD) * ae).sum(-1)
    return jnp.concatenate([c, c]).reshape(1, 16)


def _att_vec(att):
    return att.reshape(1, HC)


def kernel(x, edge_index, edge_attr, batch, hidden_state, num_graphs, params):
    p = params
    g = hidden_state.shape[1]
    hs = hidden_state.shape[2]
    na = 5
    ad = p["fc2_w"].shape[0]
    src = edge_index[0]
    dst = edge_index[1]
    ea = edge_attr.reshape(E)
    z128 = jnp.zeros((NP, HC), jnp.float32)
    z16 = jnp.zeros((NP, 16), jnp.float32)
    batch_f = batch.astype(jnp.float32).reshape(N, 1)

    hcur = x
    loop2 = None
    for li, sfx in enumerate(("1", "2", "3")):
        c16 = _layer_c16(p["le" + sfx], p["ae" + sfx])
        as2 = _att_vec(p["as" + sfx])
        ad2 = _att_vec(p["ad" + sfx])
        h, av_s, av_d = _dense_pre(hcur, p["W" + sfx], as2, ad2)
        acc2, den2 = _edge_pass(src, dst, ea, c16.reshape(16), av_s, av_d,
                                h, z128, z16)
        if li == 0:
            loop2 = den2
        hcur = _finalize(acc2, den2, h, av_s, av_d, loop2, c16,
                         p["b" + sfx].reshape(1, HC), self_loops=(li > 0))

    mean40, h_last = _head(hcur, batch_f, hidden_state[0], p, g, na, hs, ad)
    mean = mean40.reshape(g, na, ad)
    log_std = jnp.clip(p["log_std"], -20.0, 2.0)
    std = jnp.broadcast_to(jnp.exp(log_std), mean.shape)
    return mean, std, h_last[None, :, :]


# speedup vs baseline: 1.0085x; 1.0085x over previous
import functools
import jax
import jax.numpy as jnp
from jax import lax
from jax.experimental import pallas as pl
from jax.experimental.pallas import tpu as pltpu
from jax.experimental.pallas import tpu_sc as plsc

N = 10000
NP = 10240
E = 320000
HC = 128
NH = 8
HD = 16
NC = 2
NS = 16
NW = NC * NS
NBUF = 5
K = 80
BN = 400


def _dense_pre_body(p_ref, w_ref, as_ref, ad_ref, h_ref, oas_ref, oad_ref):
    h = jnp.dot(p_ref[...], w_ref[...], preferred_element_type=jnp.float32)
    h_ref[...] = h
    a_s = (h * as_ref[...]).reshape(BN, NH, HD).sum(-1)
    a_d = (h * ad_ref[...]).reshape(BN, NH, HD).sum(-1)
    oas_ref[...] = jnp.concatenate([a_s, a_s], axis=1)
    oad_ref[...] = jnp.concatenate([a_d, a_d], axis=1)


def _dense_pre(p, w, as2, ad2):
    grid = (N // BN,)
    return pl.pallas_call(
        _dense_pre_body,
        grid=grid,
        in_specs=[
            pl.BlockSpec((BN, HC), lambda i: (i, 0)),
            pl.BlockSpec((HC, HC), lambda i: (0, 0)),
            pl.BlockSpec((1, HC), lambda i: (0, 0)),
            pl.BlockSpec((1, HC), lambda i: (0, 0)),
        ],
        out_specs=[
            pl.BlockSpec((BN, HC), lambda i: (i, 0)),
            pl.BlockSpec((BN, 16), lambda i: (i, 0)),
            pl.BlockSpec((BN, 16), lambda i: (i, 0)),
        ],
        out_shape=[
            jax.ShapeDtypeStruct((NP, HC), jnp.float32),
            jax.ShapeDtypeStruct((NP, 16), jnp.float32),
            jax.ShapeDtypeStruct((NP, 16), jnp.float32),
        ],
    )(p, w, as2, ad2)


def _expand_heads(v8, rows):
    return jnp.broadcast_to(v8[:, :, None], (rows, NH, HD)).reshape(rows, HC)


def _finalize_body(self_loops, acc_ref, den_ref, h_ref, as_ref, ad_ref,
                   loop_ref, c_ref, b_ref, out_ref):
    acc = acc_ref[0] + acc_ref[1]
    den16 = den_ref[0] + den_ref[1]
    if self_loops:
        ls = loop_ref[0] + loop_ref[1]
        ea_mean = ls[:, 8:9] / jnp.maximum(ls[:, 9:10], 1.0)
        al = as_ref[...] + ad_ref[...] + ea_mean * c_ref[...]
        al = jnp.maximum(al, 0.2 * al)
        ws = jnp.exp(al)
        den16 = den16 + ws
        acc = acc + h_ref[...] * _expand_heads(ws[:, :NH], BN)
    den = _expand_heads(den16[:, :NH], BN)
    out_ref[...] = jnp.maximum(acc / (den + 1e-16) + b_ref[...], 0.0)


def _finalize(acc2, den2, h, as2, ad2, loop2, c16, bias, self_loops):
    grid = (N // BN,)
    body = functools.partial(_finalize_body, self_loops)
    return pl.pallas_call(
        body,
        grid=grid,
        in_specs=[
            pl.BlockSpec((2, BN, HC), lambda i: (0, i, 0)),
            pl.BlockSpec((2, BN, 16), lambda i: (0, i, 0)),
            pl.BlockSpec((BN, HC), lambda i: (i, 0)),
            pl.BlockSpec((BN, 16), lambda i: (i, 0)),
            pl.BlockSpec((BN, 16), lambda i: (i, 0)),
            pl.BlockSpec((2, BN, 16), lambda i: (0, i, 0)),
            pl.BlockSpec((1, 16), lambda i: (0, 0)),
            pl.BlockSpec((1, HC), lambda i: (0, 0)),
        ],
        out_specs=pl.BlockSpec((BN, HC), lambda i: (i, 0)),
        out_shape=jax.ShapeDtypeStruct((N, HC), jnp.float32),
    )(acc2, den2, h, as2, ad2, loop2, c16, bias)


def _fused_fin_pre_body(self_loops, acc_ref, den_ref, hp_ref, asp_ref,
                        adp_ref, loop_ref, cp_ref, bp_ref, w_ref, as_ref,
                        ad_ref, h_ref, oas_ref, oad_ref):
    acc = acc_ref[0] + acc_ref[1]
    den16 = den_ref[0] + den_ref[1]
    if self_loops:
        ls = loop_ref[0] + loop_ref[1]
        ea_mean = ls[:, 8:9] / jnp.maximum(ls[:, 9:10], 1.0)
        al = asp_ref[...] + adp_ref[...] + ea_mean * cp_ref[...]
        al = jnp.maximum(al, 0.2 * al)
        ws = jnp.exp(al)
        den16 = den16 + ws
        acc = acc + hp_ref[...] * _expand_heads(ws[:, :NH], BN)
    den = _expand_heads(den16[:, :NH], BN)
    fin = jnp.maximum(acc / (den + 1e-16) + bp_ref[...], 0.0)
    h = jnp.dot(fin, w_ref[...], preferred_element_type=jnp.float32)
    h_ref[...] = h
    a_s = (h * as_ref[...]).reshape(BN, NH, HD).sum(-1)
    a_d = (h * ad_ref[...]).reshape(BN, NH, HD).sum(-1)
    oas_ref[...] = jnp.concatenate([a_s, a_s], axis=1)
    oad_ref[...] = jnp.concatenate([a_d, a_d], axis=1)


def _fused_fin_pre(acc2, den2, hp, asp, adp, loop2, c16, bias, w, as2, ad2,
                   self_loops):
    grid = (N // BN,)
    body = functools.partial(_fused_fin_pre_body, self_loops)
    return pl.pallas_call(
        body,
        grid=grid,
        in_specs=[
            pl.BlockSpec((2, BN, HC), lambda i: (0, i, 0)),
            pl.BlockSpec((2, BN, 16), lambda i: (0, i, 0)),
            pl.BlockSpec((BN, HC), lambda i: (i, 0)),
            pl.BlockSpec((BN, 16), lambda i: (i, 0)),
            pl.BlockSpec((BN, 16), lambda i: (i, 0)),
            pl.BlockSpec((2, BN, 16), lambda i: (0, i, 0)),
            pl.BlockSpec((1, 16), lambda i: (0, 0)),
            pl.BlockSpec((1, HC), lambda i: (0, 0)),
            pl.BlockSpec((HC, HC), lambda i: (0, 0)),
            pl.BlockSpec((1, HC), lambda i: (0, 0)),
            pl.BlockSpec((1, HC), lambda i: (0, 0)),
        ],
        out_specs=[
            pl.BlockSpec((BN, HC), lambda i: (i, 0)),
            pl.BlockSpec((BN, 16), lambda i: (i, 0)),
            pl.BlockSpec((BN, 16), lambda i: (i, 0)),
        ],
        out_shape=[
            jax.ShapeDtypeStruct((NP, HC), jnp.float32),
            jax.ShapeDtypeStruct((NP, 16), jnp.float32),
            jax.ShapeDtypeStruct((NP, 16), jnp.float32),
        ],
    )(acc2, den2, hp, asp, adp, loop2, c16, bias, w, as2, ad2)


def _edge_body(src_hbm, dst_hbm, ea_hbm, c_hbm, as_hbm, ad_hbm,
               h_hbm, z128_hbm, z16_hbm,
               acc_out, den_out,
               srcb, dstb, eab, c_v, as_r, ad_r, h_r, w_den,
               acc_sh, den_sh, sem_i, sem_g, sem_s):
    s = lax.axis_index("s")
    c = lax.axis_index("c")
    wid = s * NC + c
    rows_per = NP // NS
    rbase = s * rows_per
    pltpu.sync_copy(z128_hbm.at[pl.ds(rbase, rows_per)],
                    acc_sh.at[pl.ds(rbase, rows_per)])
    pltpu.sync_copy(z16_hbm.at[pl.ds(rbase, rows_per)],
                    den_sh.at[pl.ds(rbase, rows_per)])
    plsc.subcore_barrier()
    pltpu.sync_copy(c_hbm, c_v)
    cvec = c_v[...]
    ew = E // NW
    ebase = wid * ew
    niter = ew // K
    lanes = lax.iota(jnp.int32, 16)

    def issue_idx(it, b):
        off = ebase + it * K
        pltpu.async_copy(src_hbm.at[pl.ds(off, K)], srcb.at[b], sem_i.at[b])
        pltpu.async_copy(dst_hbm.at[pl.ds(off, K)], dstb.at[b], sem_i.at[b])
        pltpu.async_copy(ea_hbm.at[pl.ds(off, K)], eab.at[b], sem_i.at[b])

    def wait_idx(b):
        pltpu.make_async_copy(src_hbm.at[pl.ds(0, K)], srcb.at[b],
                              sem_i.at[b]).wait()
        pltpu.make_async_copy(dst_hbm.at[pl.ds(0, K)], dstb.at[b],
                              sem_i.at[b]).wait()
        pltpu.make_async_copy(ea_hbm.at[pl.ds(0, K)], eab.at[b],
                              sem_i.at[b]).wait()

    def issue_gather_ad(b4, b2):
        pltpu.async_copy(as_hbm.at[srcb.at[b4]], as_r.at[b2], sem_g.at[b2])
        pltpu.async_copy(ad_hbm.at[dstb.at[b4]], ad_r.at[b2], sem_g.at[b2])

    def issue_gather_h(b4, b2):
        pltpu.async_copy(h_hbm.at[srcb.at[b4]], h_r.at[b2], sem_g.at[b2])

    def wait_gather(b4, b2):
        pltpu.make_async_copy(as_hbm.at[srcb.at[b4]], as_r.at[b2],
                              sem_g.at[b2]).wait()
        pltpu.make_async_copy(ad_hbm.at[dstb.at[b4]], ad_r.at[b2],
                              sem_g.at[b2]).wait()
        pltpu.make_async_copy(h_hbm.at[srcb.at[b4]], h_r.at[b2],
                              sem_g.at[b2]).wait()

    def issue_scatter(b4, b2):
        pltpu.async_copy(w_den.at[b4], den_sh.at[dstb.at[b4]], sem_s.at[b4],
                         add=True)
        pltpu.async_copy(h_r.at[b2], acc_sh.at[dstb.at[b4]], sem_s.at[b4],
                         add=True)

    def wait_scatter(b4, b2):
        pltpu.make_async_copy(w_den.at[b4], den_sh.at[dstb.at[b4]],
                              sem_s.at[b4]).wait()
        pltpu.make_async_copy(h_r.at[b2], acc_sh.at[dstb.at[b4]],
                              sem_s.at[b4]).wait()

    dnums = lax.GatherDimensionNumbers(offset_dims=(),
                                        collapsed_slice_dims=(0,),
                                        start_index_map=(0,))

    def bcast(v, lane):
        idx = (lanes * 0 + lane).reshape(16, 1)
        return lax.gather(v, idx, dnums, (1,),
                          mode=lax.GatherScatterMode.PROMISE_IN_BOUNDS)

    def compute(b4, b2):
        def blk16(jb, carry2):
            eb = eab[b4, pl.ds(jb * 16, 16)]
            for i in range(16):
                j = jb * 16 + i
                eas = eb[i]
                eav = bcast(eb, i)
                al = as_r[b2, j] + ad_r[b2, j] + eav * cvec
                al = jnp.maximum(al, 0.2 * al)
                w = jnp.exp(al)
                w_den[b4, j] = jnp.where(
                    lanes < 8, w,
                    jnp.where(lanes == 8, eas,
                              jnp.where(lanes == 9, 1.0, 0.0)))
                for hh in range(NH):
                    sl = pl.ds(hh * HD, HD)
                    h_r[b2, j, sl] = h_r[b2, j, sl] * bcast(w, hh)
            return carry2

        lax.fori_loop(0, K // 16, blk16, 0)

    def body(it, b0):
        b2 = b0 % 2
        bp1_4 = (b0 + 1) % 4
        bp1_2 = (b0 + 1) % 2
        bp2_4 = (b0 + 2) % 4

        @pl.when(it + 2 < niter)
        def _():
            issue_idx(it + 2, bp2_4)

        @pl.when(it + 1 < niter)
        def _():
            wait_idx(bp1_4)
            issue_gather_ad(bp1_4, bp1_2)

        wait_gather(b0 % 4, b2)
        compute(b0 % 4, b2)

        @pl.when(it >= 1)
        def _():
            wait_scatter((b0 + 3) % 4, bp1_2)

        @pl.when(it + 1 < niter)
        def _():
            issue_gather_h(bp1_4, bp1_2)

        issue_scatter(b0 % 4, b2)

    issue_idx(0, 0)
    issue_idx(1, 1)
    wait_idx(0)
    issue_gather_ad(0, 0)
    issue_gather_h(0, 0)

    def outer(g, carry):
        for b0 in range(4):
            body(g * 4 + b0, b0)
        return carry

    lax.fori_loop(0, niter // 4, outer, 0)
    body(niter - 1, 0)
    wait_scatter(0, 0)
    plsc.subcore_barrier()
    obase = c * NP + rbase
    pltpu.sync_copy(acc_sh.at[pl.ds(rbase, rows_per)],
                    acc_out.at[pl.ds(obase, rows_per)])
    pltpu.sync_copy(den_sh.at[pl.ds(rbase, rows_per)],
                    den_out.at[pl.ds(obase, rows_per)])


def _edge_pass(src, dst, ea, c16, as2, ad2, h, z128, z16):
    mesh = plsc.VectorSubcoreMesh(core_axis_name="c", subcore_axis_name="s",
                                  num_cores=NC, num_subcores=NS)
    out_type = [
        jax.ShapeDtypeStruct((NC * NP, HC), jnp.float32),
        jax.ShapeDtypeStruct((NC * NP, 16), jnp.float32),
    ]
    scratch = [
        pltpu.VMEM((4, K), jnp.int32),
        pltpu.VMEM((4, K), jnp.int32),
        pltpu.VMEM((4, K), jnp.float32),
        pltpu.VMEM((16,), jnp.float32),
        pltpu.VMEM((2, K, 16), jnp.float32),
        pltpu.VMEM((2, K, 16), jnp.float32),
        pltpu.VMEM((2, K, HC), jnp.float32),
        pltpu.VMEM((4, K, 16), jnp.float32),
        pltpu.VMEM_SHARED((NP, HC), jnp.float32),
        pltpu.VMEM_SHARED((NP, 16), jnp.float32),
        pltpu.SemaphoreType.DMA((4,)),
        pltpu.SemaphoreType.DMA((2,)),
        pltpu.SemaphoreType.DMA((4,)),
    ]
    fn = pl.kernel(_edge_body, out_type=out_type, mesh=mesh,
                   scratch_types=scratch,
                   compiler_params=pltpu.CompilerParams(
                       use_tc_tiling_on_sc=False))
    acc2, den2 = fn(src, dst, ea, c16, as2, ad2, h, z128, z16)
    return acc2.reshape(NC, NP, HC), den2.reshape(NC, NP, 16)


def _sigmoid(x):
    return 1.0 / (1.0 + jnp.exp(-x))


def _head_body(na, g, hs, ad, h_ref, b_ref, hid_ref, wih_ref, whh_ref,
               bih_ref, bhh_ref, f1w_ref, f1b_ref, f2w_ref, f2b_ref,
               mean_ref, hl_ref, agent_ref):
    gvec = lax.broadcasted_iota(jnp.int32, (1, g), 1).astype(jnp.float32)
    cb = 200

    def blk(i, carry):
        sums, cnt, ltc = carry
        bblk = b_ref[pl.ds(i * cb, cb), :]
        hblk = h_ref[pl.ds(i * cb, cb), :]
        oh = jnp.where(bblk == gvec, 1.0, 0.0)
        lt = jnp.where(bblk < gvec, 1.0, 0.0)
        sums = sums + lax.dot_general(oh, hblk, (((0,), (0,)), ((), ())),
                                      preferred_element_type=jnp.float32)
        cnt = cnt + jnp.sum(oh, axis=0)
        ltc = ltc + jnp.sum(lt, axis=0)
        return sums, cnt, ltc

    sums, cnt, ltc = lax.fori_loop(
        0, N // cb, blk,
        (jnp.zeros((g, HC), jnp.float32), jnp.zeros((g,), jnp.float32),
         jnp.zeros((g,), jnp.float32)))
    ge = sums / jnp.maximum(cnt, 1.0)[:, None]
    starts = ltc.astype(jnp.int32)
    for gg in range(g):
        sg = starts[gg]
        for a in range(na):
            idx = jnp.minimum(sg + a, N - 1)
            agent_ref[pl.ds(a * g + gg, 1), :] = h_ref[pl.ds(idx, 1), :]
    hcur = hid_ref[...]
    ys = []
    for t in range(na):
        xa = agent_ref[pl.ds(t * g, g), :]
        xt = jnp.concatenate([xa, ge], axis=1)
        gi = lax.dot_general(xt, wih_ref[...], (((1,), (1,)), ((), ())),
                             preferred_element_type=jnp.float32) + bih_ref[...]
        gh = lax.dot_general(hcur, whh_ref[...], (((1,), (1,)), ((), ())),
                             preferred_element_type=jnp.float32) + bhh_ref[...]
        r = _sigmoid(gi[:, :hs] + gh[:, :hs])
        z = _sigmoid(gi[:, hs:2 * hs] + gh[:, hs:2 * hs])
        nt = jnp.tanh(gi[:, 2 * hs:] + r * gh[:, 2 * hs:])
        hcur = (1.0 - z) * nt + z * hcur
        ys.append(hcur)
    gru = jnp.stack(ys, axis=1).reshape(g * na, hs)
    f1 = jnp.maximum(
        lax.dot_general(gru, f1w_ref[...], (((1,), (1,)), ((), ())),
                        preferred_element_type=jnp.float32) + f1b_ref[...],
        0.0)
    mo = lax.dot_general(f1, f2w_ref[...], (((1,), (1,)), ((), ())),
                         preferred_element_type=jnp.float32) + f2b_ref[...]
    mean_ref[...] = mo
    hl_ref[...] = hcur


def _head(h, batch_f, hidden, p, g, na, hs, ad):
    body = functools.partial(_head_body, na, g, hs, ad)
    return pl.pallas_call(
        body,
        out_shape=[
            jax.ShapeDtypeStruct((g * na, ad), jnp.float32),
            jax.ShapeDtypeStruct((g, hs), jnp.float32),
        ],
        scratch_shapes=[pltpu.VMEM((g * na, HC), jnp.float32)],
    )(h, batch_f, hidden, p["w_ih"], p["w_hh"], p["b_ih"].reshape(1, -1),
      p["b_hh"].reshape(1, -1), p["fc1_w"], p["fc1_b"].reshape(1, -1),
      p["fc2_w"], p["fc2_b"].reshape(1, -1))


def _layer_c16(le, ae):
    c = (le.reshape(NH, HD) * ae).sum(-1)
    return jnp.concatenate([c, c]).reshape(1, 16)


def _att_vec(att):
    return att.reshape(1, HC)


def kernel(x, edge_index, edge_attr, batch, hidden_state, num_graphs, params):
    p = params
    g = hidden_state.shape[1]
    hs = hidden_state.shape[2]
    na = 5
    ad = p["fc2_w"].shape[0]
    src = edge_index[0]
    dst = edge_index[1]
    ea = edge_attr.reshape(E)
    z128 = jnp.zeros((NP, HC), jnp.float32)
    z16 = jnp.zeros((NP, 16), jnp.float32)
    batch_f = batch.astype(jnp.float32).reshape(N, 1)

    c16s = [_layer_c16(p["le" + s_], p["ae" + s_]) for s_ in ("1", "2", "3")]
    asvs = [_att_vec(p["as" + s_]) for s_ in ("1", "2", "3")]
    advs = [_att_vec(p["ad" + s_]) for s_ in ("1", "2", "3")]
    bs = [p["b" + s_].reshape(1, HC) for s_ in ("1", "2", "3")]
    ws = [p["W" + s_] for s_ in ("1", "2", "3")]

    h, av_s, av_d = _dense_pre(x, ws[0], asvs[0], advs[0])
    acc2, den2 = _edge_pass(src, dst, ea, c16s[0].reshape(16), av_s, av_d,
                            h, z128, z16)
    loop2 = den2
    for li in (1, 2):
        h, av_s, av_d = _fused_fin_pre(acc2, den2, h, av_s, av_d, loop2,
                                       c16s[li - 1], bs[li - 1], ws[li],
                                       asvs[li], advs[li],
                                       self_loops=(li - 1 > 0))
        acc2, den2 = _edge_pass(src, dst, ea, c16s[li].reshape(16), av_s,
                                av_d, h, z128, z16)
    hcur = _finalize(acc2, den2, h, av_s, av_d, loop2, c16s[2], bs[2],
                     self_loops=True)

    mean40, h_last = _head(hcur, batch_f, hidden_state[0], p, g, na, hs, ad)
    mean = mean40.reshape(g, na, ad)
    log_std = jnp.clip(p["log_std"], -20.0, 2.0)
    std = jnp.broadcast_to(jnp.exp(log_std), mean.shape)
    return mean, std, h_last[None, :, :]
